# Initial kernel scaffold; baseline (speedup 1.0000x reference)
import jax
import jax.numpy as jnp
from jax.experimental import pallas as pl


def kernel(nodes, globals_, senders, receivers, params):
    raise NotImplementedError("write your pallas kernel here")


# speedup vs baseline: 1.8935x; 1.8935x over previous
import functools

import jax
import jax.numpy as jnp
from jax import lax
from jax.experimental import pallas as pl
from jax.experimental.pallas import tpu as pltpu
from jax.experimental.pallas import tpu_sc as plsc

N = 10000
N_PAD = 10240
E = 160000
LAT = 128
NC = 2
NS = 16
NW = NC * NS
C = 64
NCHUNK = E // C
ROWS_PER_TILE = N_PAD // NS
AUG = LAT + 16

_f32 = jnp.float32


def _mm(x, w):
    return jnp.dot(x, w, preferred_element_type=_f32)


def _tc_pre(nodes, wemb, bemb, ws, c_s, wr):
    blk, grid = 2000, N // 2000

    def kern(nref, wE, bE, wS, cS, wR, xo, ao, bo):
        xb = _mm(nref[...], wE[...]) + bE[...]
        xo[...] = xb
        ao[...] = _mm(xb, wS[...]) + cS[...]
        bo[...] = _mm(xb, wR[...])

    row = pl.BlockSpec((blk, LAT), lambda i: (i, 0))
    full = pl.BlockSpec((LAT, LAT), lambda i: (0, 0))
    vec = pl.BlockSpec((1, LAT), lambda i: (0, 0))
    out = jax.ShapeDtypeStruct((N, LAT), _f32)
    return pl.pallas_call(
        kern, grid=(grid,),
        in_specs=[pl.BlockSpec((blk, nodes.shape[1]), lambda i: (i, 0)),
                  pl.BlockSpec((nodes.shape[1], LAT), lambda i: (0, 0)),
                  vec, full, vec, full],
        out_specs=[row, row, row],
        out_shape=[out, out, out],
    )(nodes, wemb, bemb.reshape(1, LAT), ws, c_s.reshape(1, LAT), wr)


def _tc_edge_transform(h0, wec_pad):
    blk, grid = 2000, E // 2000

    def kern(href, w, o):
        o[...] = _mm(href[...], w[...])

    return pl.pallas_call(
        kern, grid=(grid,),
        in_specs=[pl.BlockSpec((blk, AUG), lambda i: (i, 0)),
                  pl.BlockSpec((AUG, LAT), lambda i: (0, 0))],
        out_specs=pl.BlockSpec((blk, LAT), lambda i: (i, 0)),
        out_shape=jax.ShapeDtypeStruct((E, LAT), _f32),
    )(h0, wec_pad)


def _tc_node(x, s_parts, A, B_aug, c, wn2, bn2, scale, bias,
             tail_ws=None, tail_c=None, tail_wr=None, dec_w=None, dec_b=None):
    blk, grid = 2000, N // 2000
    final = dec_w is not None

    def kern(xref, sref, Ar, Br, cr, w2r, b2r, scr, bir, w1r, c1r, w2b, *o):
        xb = xref[...]
        s = sref[0] + sref[1]
        hid = jnp.maximum(_mm(xb, Ar[...]) + _mm(s, Br[...]) + cr[...], 0.0)
        y = xb + _mm(hid, w2r[...]) + b2r[...]
        m = jnp.mean(y, axis=-1, keepdims=True)
        yc = y - m
        var = jnp.mean(yc * yc, axis=-1, keepdims=True)
        xn = yc * lax.rsqrt(var + 1e-6) * scr[...] + bir[...]
        if final:
            o[0][...] = _mm(xn, w1r[...]) + c1r[...]
        else:
            o[0][...] = xn
            o[1][...] = _mm(xn, w1r[...]) + c1r[...]
            o[2][...] = _mm(xn, w2b[...])

    row = pl.BlockSpec((blk, LAT), lambda i: (i, 0))
    full = pl.BlockSpec((LAT, LAT), lambda i: (0, 0))
    vec = pl.BlockSpec((1, LAT), lambda i: (0, 0))
    out = jax.ShapeDtypeStruct((N, LAT), _f32)
    if final:
        w1, c1, w2b = dec_w, dec_b.reshape(1, LAT), jnp.zeros((LAT, LAT), _f32)
        out_specs, out_shape = [row], [out]
    else:
        w1, c1, w2b = tail_ws, tail_c.reshape(1, LAT), tail_wr
        out_specs, out_shape = [row, row, row], [out, out, out]
    res = pl.pallas_call(
        kern, grid=(grid,),
        in_specs=[row,
                  pl.BlockSpec((2, blk, AUG), lambda i: (0, i, 0)),
                  full, pl.BlockSpec((AUG, LAT), lambda i: (0, 0)),
                  vec, full, vec, vec, vec, full, vec, full],
        out_specs=out_specs, out_shape=out_shape,
    )(x, s_parts, A, B_aug, c.reshape(1, LAT),
      wn2, bn2.reshape(1, LAT), scale.reshape(1, LAT), bias.reshape(1, LAT),
      w1, c1, w2b)
    return res[0] if final else res


def _sc_edge_step(with_h_out):
    mesh = plsc.VectorSubcoreMesh(core_axis_name="c", subcore_axis_name="s",
                                  num_cores=NC, num_subcores=NS)
    s_out_t = jax.ShapeDtypeStruct((NC, N_PAD, AUG), _f32)
    if with_h_out:
        out_type = [jax.ShapeDtypeStruct((E, AUG), _f32), s_out_t]
    else:
        out_type = [s_out_t]
    scratch = [
        pltpu.VMEM((C,), jnp.int32),
        pltpu.VMEM((C,), jnp.int32),
        pltpu.VMEM((C, LAT), _f32),
        pltpu.VMEM((C, LAT), _f32),
        pltpu.VMEM((C, LAT), _f32),
        pltpu.VMEM((C, AUG), _f32),
        pltpu.SemaphoreType.DMA,
        pltpu.SemaphoreType.DMA,
        pltpu.VMEM_SHARED((N_PAD, AUG), _f32),
    ]

    def body(*refs):
        if with_h_out:
            (a_hbm, b_hbm, snd_hbm, rcv_hbm, h_out, s_out,
             sidx, ridx, arows, brows, hbrows, hbuf, sem1, sem2, acc) = refs
        else:
            (a_hbm, b_hbm, snd_hbm, rcv_hbm, hb_hbm, s_out,
             sidx, ridx, arows, brows, hbrows, hbuf, sem1, sem2, acc) = refs
        cid = lax.axis_index("c")
        sid = lax.axis_index("s")
        wid = cid * NS + sid
        row0 = sid * ROWS_PER_TILE

        zv = jnp.zeros((16,), _f32)
        ov = jnp.ones((16,), _f32)

        def fill_z(i, _):
            for j in range(AUG // 16):
                hbuf[i, pl.ds(j * 16, 16)] = zv
            return 0
        lax.fori_loop(0, C, fill_z, 0)

        def zero_acc(k, _):
            pltpu.sync_copy(hbuf, acc.at[pl.ds(row0 + k * C, C)])
            return 0
        lax.fori_loop(0, ROWS_PER_TILE // C, zero_acc, 0)

        def fill_o(i, _):
            hbuf[i, pl.ds(LAT, 16)] = ov
            return 0
        lax.fori_loop(0, C, fill_o, 0)
        plsc.subcore_barrier()

        n_chunks = jnp.int32(NCHUNK // NW) + jnp.where(wid < NCHUNK % NW, 1, 0)

        def chunk(i, _):
            base = (wid + NW * i) * C
            pltpu.sync_copy(snd_hbm.at[pl.ds(base, C)], sidx)
            pltpu.sync_copy(rcv_hbm.at[pl.ds(base, C)], ridx)
            cp1 = pltpu.async_copy(a_hbm.at[sidx], arows, sem1)
            cp2 = pltpu.async_copy(b_hbm.at[ridx], brows, sem2)
            if not with_h_out:
                pltpu.sync_copy(hb_hbm.at[pl.ds(base, C)], hbrows)
            cp1.wait()
            cp2.wait()

            def row(rr, _):
                for j in range(LAT // 16):
                    sl = pl.ds(j * 16, 16)
                    acc_v = arows[rr, sl] + brows[rr, sl]
                    if not with_h_out:
                        acc_v = acc_v + hbrows[rr, sl]
                    hbuf[rr, sl] = jnp.maximum(acc_v, 0.0)
                return 0
            lax.fori_loop(0, C, row, 0)

            if with_h_out:
                pltpu.sync_copy(hbuf, h_out.at[pl.ds(base, C)])
            pltpu.sync_copy(hbuf, acc.at[ridx], add=True)
            return 0
        lax.fori_loop(0, n_chunks, chunk, 0)
        plsc.subcore_barrier()

        def readout(k, _):
            pltpu.sync_copy(acc.at[pl.ds(row0 + k * C, C)],
                            s_out.at[cid, pl.ds(row0 + k * C, C)])
            return 0
        lax.fori_loop(0, ROWS_PER_TILE // C, readout, 0)

    return pl.kernel(body, out_type=out_type, mesh=mesh, scratch_types=scratch,
                     compiler_params=pltpu.CompilerParams(use_tc_tiling_on_sc=False))


def kernel(nodes, globals_, senders, receivers, params):
    gv = globals_.reshape(1, -1).astype(_f32)

    (w1e0, b1e0), (w2e0, b2e0) = params["edge_mlp_0"]
    ws0, wr0, wg0 = w1e0[:LAT], w1e0[LAT:2 * LAT], w1e0[2 * LAT:]
    c0 = (gv @ wg0)[0] + b1e0

    (w1e1, b1e1), (w2e1, b2e1) = params["edge_mlp_1"]
    we1, ws1, wr1, wg1 = (w1e1[:LAT], w1e1[LAT:2 * LAT],
                          w1e1[2 * LAT:3 * LAT], w1e1[3 * LAT:])
    wec_pad = jnp.concatenate([w2e0 @ we1, jnp.zeros((AUG - LAT, LAT), _f32)], 0)
    c1 = b2e0 @ we1 + (gv @ wg1)[0] + b1e1

    def node_params(step):
        (wn1, bn1), (wn2, bn2) = params["node_mlp_%d" % step]
        w2e, b2e = params["edge_mlp_%d" % step][1]
        wnr = wn1[LAT:2 * LAT]
        A = wn1[:LAT]
        B_aug = jnp.concatenate(
            [w2e @ wnr, (b2e @ wnr)[None, :],
             jnp.zeros((AUG - LAT - 1, LAT), _f32)], 0)
        c = (gv @ wn1[2 * LAT:])[0] + bn1
        sc, bi = params["ln_%d" % step]
        return A, B_aug, c, wn2, bn2, sc, bi

    wemb, bemb = params["emb"]
    wdec, bdec = params["dec"]

    x0, a0, b0 = _tc_pre(nodes, wemb, bemb, ws0, c0, wr0)
    h0, s0 = _sc_edge_step(True)(a0, b0, senders, receivers)
    hb = _tc_edge_transform(h0, wec_pad)
    x1, a1, b1 = _tc_node(x0, s0, *node_params(0),
                          tail_ws=ws1, tail_c=c1, tail_wr=wr1)
    (s1,) = _sc_edge_step(False)(a1, b1, senders, receivers, hb)
    out = _tc_node(x1, s1, *node_params(1), dec_w=wdec, dec_b=bdec)
    return out


# speedup vs baseline: 2.0967x; 1.1073x over previous
import functools

import jax
import jax.numpy as jnp
from jax import lax
from jax.experimental import pallas as pl
from jax.experimental.pallas import tpu as pltpu
from jax.experimental.pallas import tpu_sc as plsc

N = 10000
N_PAD = 10240
E = 160000
LAT = 128
NC = 2
NS = 16
NW = NC * NS
C = 40
NCHUNK = E // C
CPW = NCHUNK // NW
ROWS_PER_TILE = N_PAD // NS
AUG = LAT + 16

_f32 = jnp.float32


def _mm(x, w):
    return jnp.dot(x, w, preferred_element_type=_f32)


def _tc_pre(nodes, wemb, bemb, ws, c_s, wr):
    blk, grid = 2000, N // 2000

    def kern(nref, wE, bE, wS, cS, wR, xo, ao, bo):
        xb = _mm(nref[...], wE[...]) + bE[...]
        xo[...] = xb
        ao[...] = _mm(xb, wS[...]) + cS[...]
        bo[...] = _mm(xb, wR[...])

    row = pl.BlockSpec((blk, LAT), lambda i: (i, 0))
    full = pl.BlockSpec((LAT, LAT), lambda i: (0, 0))
    vec = pl.BlockSpec((1, LAT), lambda i: (0, 0))
    out = jax.ShapeDtypeStruct((N, LAT), _f32)
    return pl.pallas_call(
        kern, grid=(grid,),
        in_specs=[pl.BlockSpec((blk, nodes.shape[1]), lambda i: (i, 0)),
                  pl.BlockSpec((nodes.shape[1], LAT), lambda i: (0, 0)),
                  vec, full, vec, full],
        out_specs=[row, row, row],
        out_shape=[out, out, out],
    )(nodes, wemb, bemb.reshape(1, LAT), ws, c_s.reshape(1, LAT), wr)


def _tc_edge_transform(h0, wec_pad):
    blk, grid = 2000, E // 2000

    def kern(href, w, o):
        o[...] = _mm(href[...], w[...])

    return pl.pallas_call(
        kern, grid=(grid,),
        in_specs=[pl.BlockSpec((blk, AUG), lambda i: (i, 0)),
                  pl.BlockSpec((AUG, LAT), lambda i: (0, 0))],
        out_specs=pl.BlockSpec((blk, LAT), lambda i: (i, 0)),
        out_shape=jax.ShapeDtypeStruct((E, LAT), _f32),
    )(h0, wec_pad)


def _tc_node(x, s_parts, A, B_aug, c, wn2, bn2, scale, bias,
             tail_ws=None, tail_c=None, tail_wr=None, dec_w=None, dec_b=None):
    blk, grid = 2000, N // 2000
    final = dec_w is not None

    def kern(xref, sref, Ar, Br, cr, w2r, b2r, scr, bir, w1r, c1r, w2b, *o):
        xb = xref[...]
        s = sref[0] + sref[1]
        hid = jnp.maximum(_mm(xb, Ar[...]) + _mm(s, Br[...]) + cr[...], 0.0)
        y = xb + _mm(hid, w2r[...]) + b2r[...]
        m = jnp.mean(y, axis=-1, keepdims=True)
        yc = y - m
        var = jnp.mean(yc * yc, axis=-1, keepdims=True)
        xn = yc * lax.rsqrt(var + 1e-6) * scr[...] + bir[...]
        if final:
            o[0][...] = _mm(xn, w1r[...]) + c1r[...]
        else:
            o[0][...] = xn
            o[1][...] = _mm(xn, w1r[...]) + c1r[...]
            o[2][...] = _mm(xn, w2b[...])

    row = pl.BlockSpec((blk, LAT), lambda i: (i, 0))
    full = pl.BlockSpec((LAT, LAT), lambda i: (0, 0))
    vec = pl.BlockSpec((1, LAT), lambda i: (0, 0))
    out = jax.ShapeDtypeStruct((N, LAT), _f32)
    if final:
        w1, c1, w2b = dec_w, dec_b.reshape(1, LAT), jnp.zeros((LAT, LAT), _f32)
        out_specs, out_shape = [row], [out]
    else:
        w1, c1, w2b = tail_ws, tail_c.reshape(1, LAT), tail_wr
        out_specs, out_shape = [row, row, row], [out, out, out]
    res = pl.pallas_call(
        kern, grid=(grid,),
        in_specs=[row,
                  pl.BlockSpec((2, blk, AUG), lambda i: (0, i, 0)),
                  full, pl.BlockSpec((AUG, LAT), lambda i: (0, 0)),
                  vec, full, vec, vec, vec, full, vec, full],
        out_specs=out_specs, out_shape=out_shape,
    )(x, s_parts, A, B_aug, c.reshape(1, LAT),
      wn2, bn2.reshape(1, LAT), scale.reshape(1, LAT), bias.reshape(1, LAT),
      w1, c1, w2b)
    return res[0] if final else res


def _sc_edge_step(with_h_out):
    mesh = plsc.VectorSubcoreMesh(core_axis_name="c", subcore_axis_name="s",
                                  num_cores=NC, num_subcores=NS)
    s_out_t = jax.ShapeDtypeStruct((NC, N_PAD, AUG), _f32)
    if with_h_out:
        out_type = [jax.ShapeDtypeStruct((E, AUG), _f32), s_out_t]
    else:
        out_type = [s_out_t]
    slot_scratch = [
        pltpu.VMEM((C,), jnp.int32),
        pltpu.VMEM((C,), jnp.int32),
        pltpu.VMEM((C, LAT), _f32),
        pltpu.VMEM((C, LAT), _f32),
        pltpu.SemaphoreType.DMA,
        pltpu.SemaphoreType.DMA,
    ]
    if not with_h_out:
        slot_scratch += [pltpu.VMEM((C, LAT), _f32), pltpu.SemaphoreType.DMA]
    scratch = slot_scratch * 2 + [
        pltpu.VMEM((C, AUG), _f32),
        pltpu.VMEM_SHARED((N_PAD, AUG), _f32),
    ]
    nslot = len(slot_scratch)

    def body(*refs):
        if with_h_out:
            a_hbm, b_hbm, snd_hbm, rcv_hbm, h_out, s_out = refs[:6]
            hb_hbm = None
        else:
            a_hbm, b_hbm, snd_hbm, rcv_hbm, hb_hbm, s_out = refs[:6]
        slots = [refs[6 + k * nslot: 6 + (k + 1) * nslot] for k in (0, 1)]
        hbuf, acc = refs[6 + 2 * nslot], refs[7 + 2 * nslot]
        cid = lax.axis_index("c")
        sid = lax.axis_index("s")
        wid = cid * NS + sid
        row0 = sid * ROWS_PER_TILE

        zv = jnp.zeros((16,), _f32)
        ov = jnp.ones((16,), _f32)

        def fill_z(i, _):
            for j in range(AUG // 16):
                hbuf[i, pl.ds(j * 16, 16)] = zv
            return 0
        lax.fori_loop(0, C, fill_z, 0)

        def zero_acc(k, _):
            pltpu.sync_copy(hbuf, acc.at[pl.ds(row0 + k * C, C)])
            return 0
        lax.fori_loop(0, ROWS_PER_TILE // C, zero_acc, 0)

        def fill_o(i, _):
            hbuf[i, pl.ds(LAT, 16)] = ov
            return 0
        lax.fori_loop(0, C, fill_o, 0)
        plsc.subcore_barrier()

        def gathers(slot, k):
            if with_h_out:
                sidx, ridx, arows, brows, sem_a, sem_b = slot
            else:
                sidx, ridx, arows, brows, sem_a, sem_b, hbrows, sem_h = slot
            cps = [pltpu.make_async_copy(a_hbm.at[sidx], arows, sem_a),
                   pltpu.make_async_copy(b_hbm.at[ridx], brows, sem_b)]
            if not with_h_out:
                base = (wid + NW * k) * C
                cps.append(pltpu.make_async_copy(
                    hb_hbm.at[pl.ds(base, C)], hbrows, sem_h))
            return cps

        def issue(slot, k):
            sidx, ridx = slot[0], slot[1]
            base = (wid + NW * k) * C
            pltpu.sync_copy(snd_hbm.at[pl.ds(base, C)], sidx)
            pltpu.sync_copy(rcv_hbm.at[pl.ds(base, C)], ridx)
            for cp in gathers(slot, k):
                cp.start()

        def finish(slot, k):
            if with_h_out:
                sidx, ridx, arows, brows, sem_a, sem_b = slot
                hbrows = None
            else:
                sidx, ridx, arows, brows, sem_a, sem_b, hbrows, sem_h = slot
            for cp in gathers(slot, k):
                cp.wait()

            def row(rr, _):
                for j in range(LAT // 16):
                    sl = pl.ds(j * 16, 16)
                    acc_v = arows[rr, sl] + brows[rr, sl]
                    if not with_h_out:
                        acc_v = acc_v + hbrows[rr, sl]
                    hbuf[rr, sl] = jnp.maximum(acc_v, 0.0)
                return 0
            lax.fori_loop(0, C, row, 0)

            base = (wid + NW * k) * C
            if with_h_out:
                pltpu.sync_copy(hbuf, h_out.at[pl.ds(base, C)])
            pltpu.sync_copy(hbuf, acc.at[ridx], add=True)

        issue(slots[0], 0)
        issue(slots[1], 1)

        def pair(i, _):
            finish(slots[0], 2 * i)
            issue(slots[0], 2 * i + 2)
            finish(slots[1], 2 * i + 1)

            @pl.when(i < CPW // 2 - 1)
            def _():
                issue(slots[1], 2 * i + 3)
            return 0
        lax.fori_loop(0, CPW // 2, pair, 0)
        finish(slots[0], CPW - 1)
        plsc.subcore_barrier()

        def readout(k, _):
            pltpu.sync_copy(acc.at[pl.ds(row0 + k * C, C)],
                            s_out.at[cid, pl.ds(row0 + k * C, C)])
            return 0
        lax.fori_loop(0, ROWS_PER_TILE // C, readout, 0)

    return pl.kernel(body, out_type=out_type, mesh=mesh, scratch_types=scratch,
                     compiler_params=pltpu.CompilerParams(use_tc_tiling_on_sc=False))


def kernel(nodes, globals_, senders, receivers, params):
    gv = globals_.reshape(1, -1).astype(_f32)

    (w1e0, b1e0), (w2e0, b2e0) = params["edge_mlp_0"]
    ws0, wr0, wg0 = w1e0[:LAT], w1e0[LAT:2 * LAT], w1e0[2 * LAT:]
    c0 = (gv @ wg0)[0] + b1e0

    (w1e1, b1e1), (w2e1, b2e1) = params["edge_mlp_1"]
    we1, ws1, wr1, wg1 = (w1e1[:LAT], w1e1[LAT:2 * LAT],
                          w1e1[2 * LAT:3 * LAT], w1e1[3 * LAT:])
    wec_pad = jnp.concatenate([w2e0 @ we1, jnp.zeros((AUG - LAT, LAT), _f32)], 0)
    c1 = b2e0 @ we1 + (gv @ wg1)[0] + b1e1

    def node_params(step):
        (wn1, bn1), (wn2, bn2) = params["node_mlp_%d" % step]
        w2e, b2e = params["edge_mlp_%d" % step][1]
        wnr = wn1[LAT:2 * LAT]
        A = wn1[:LAT]
        B_aug = jnp.concatenate(
            [w2e @ wnr, (b2e @ wnr)[None, :],
             jnp.zeros((AUG - LAT - 1, LAT), _f32)], 0)
        c = (gv @ wn1[2 * LAT:])[0] + bn1
        sc, bi = params["ln_%d" % step]
        return A, B_aug, c, wn2, bn2, sc, bi

    wemb, bemb = params["emb"]
    wdec, bdec = params["dec"]

    x0, a0, b0 = _tc_pre(nodes, wemb, bemb, ws0, c0, wr0)
    h0, s0 = _sc_edge_step(True)(a0, b0, senders, receivers)
    hb = _tc_edge_transform(h0, wec_pad)
    x1, a1, b1 = _tc_node(x0, s0, *node_params(0),
                          tail_ws=ws1, tail_c=c1, tail_wr=wr1)
    (s1,) = _sc_edge_step(False)(a1, b1, senders, receivers, hb)
    out = _tc_node(x1, s1, *node_params(1), dec_w=wdec, dec_b=bdec)
    return out


# speedup vs baseline: 2.8507x; 1.3596x over previous
import functools

import jax
import jax.numpy as jnp
from jax import lax
from jax.experimental import pallas as pl
from jax.experimental.pallas import tpu as pltpu
from jax.experimental.pallas import tpu_sc as plsc

N = 10000
N_PAD = 10240
E = 160000
LAT = 128
NC = 2
NS = 16
NW = NC * NS
C = 40
NCHUNK = E // C
CPW = NCHUNK // NW
ROWS_PER_TILE = N_PAD // NS
AUG = LAT + 16

_f32 = jnp.float32


def _mm(x, w):
    return jnp.dot(x, w, preferred_element_type=_f32)


def _tc_pre(nodes, wemb, bemb, ws, c_s, wr):
    blk, grid = 2000, N // 2000

    def kern(nref, wE, bE, wS, cS, wR, xo, ao, bo):
        xb = _mm(nref[...], wE[...]) + bE[...]
        xo[...] = xb
        ao[...] = _mm(xb, wS[...]) + cS[...]
        bo[...] = _mm(xb, wR[...])

    row = pl.BlockSpec((blk, LAT), lambda i: (i, 0))
    full = pl.BlockSpec((LAT, LAT), lambda i: (0, 0))
    vec = pl.BlockSpec((1, LAT), lambda i: (0, 0))
    out = jax.ShapeDtypeStruct((N, LAT), _f32)
    return pl.pallas_call(
        kern, grid=(grid,),
        in_specs=[pl.BlockSpec((blk, nodes.shape[1]), lambda i: (i, 0)),
                  pl.BlockSpec((nodes.shape[1], LAT), lambda i: (0, 0)),
                  vec, full, vec, full],
        out_specs=[row, row, row],
        out_shape=[out, out, out],
    )(nodes, wemb, bemb.reshape(1, LAT), ws, c_s.reshape(1, LAT), wr)


def _tc_edge_transform(h0, wec_pad):
    blk, grid = 2000, E // 2000

    def kern(href, w, o):
        o[...] = _mm(href[...], w[...])

    return pl.pallas_call(
        kern, grid=(grid,),
        in_specs=[pl.BlockSpec((blk, AUG), lambda i: (i, 0)),
                  pl.BlockSpec((AUG, LAT), lambda i: (0, 0))],
        out_specs=pl.BlockSpec((blk, LAT), lambda i: (i, 0)),
        out_shape=jax.ShapeDtypeStruct((E, LAT), _f32),
    )(h0, wec_pad)


def _tc_node(x, s_parts, A, B_aug, c, wn2, bn2, scale, bias,
             tail_ws=None, tail_c=None, tail_wr=None, dec_w=None, dec_b=None,
             d_parts=None, v=None):
    blk, grid = 2000, N // 2000
    final = dec_w is not None
    swidth = LAT if final else AUG

    def kern(xref, sref, *rest):
        if final:
            (dref, Ar, Br, vr, cr, w2r, b2r, scr, bir, w1r, c1r, w2b), o = rest[:12], rest[12:]
        else:
            (Ar, Br, cr, w2r, b2r, scr, bir, w1r, c1r, w2b), o = rest[:10], rest[10:]
        xb = xref[...]
        s = sref[0] + sref[1]
        hid = _mm(xb, Ar[...]) + _mm(s, Br[...]) + cr[...]
        if final:
            dg = dref[0][:, LAT:LAT + 1] + dref[1][:, LAT:LAT + 1]
            hid = hid + dg * vr[...]
        hid = jnp.maximum(hid, 0.0)
        y = xb + _mm(hid, w2r[...]) + b2r[...]
        m = jnp.mean(y, axis=-1, keepdims=True)
        yc = y - m
        var = jnp.mean(yc * yc, axis=-1, keepdims=True)
        xn = yc * lax.rsqrt(var + 1e-6) * scr[...] + bir[...]
        if final:
            o[0][...] = _mm(xn, w1r[...]) + c1r[...]
        else:
            o[0][...] = xn
            o[1][...] = _mm(xn, w1r[...]) + c1r[...]
            o[2][...] = _mm(xn, w2b[...])

    row = pl.BlockSpec((blk, LAT), lambda i: (i, 0))
    full = pl.BlockSpec((LAT, LAT), lambda i: (0, 0))
    vec = pl.BlockSpec((1, LAT), lambda i: (0, 0))
    out = jax.ShapeDtypeStruct((N, LAT), _f32)
    s_spec = pl.BlockSpec((2, blk, swidth), lambda i: (0, i, 0))
    if final:
        w1, c1, w2b = dec_w, dec_b.reshape(1, LAT), jnp.zeros((LAT, LAT), _f32)
        out_specs, out_shape = [row], [out]
        mid_specs = [pl.BlockSpec((2, blk, AUG), lambda i: (0, i, 0)),
                     full, pl.BlockSpec((swidth, LAT), lambda i: (0, 0)), vec]
        mid_args = [d_parts, A, B_aug, v.reshape(1, LAT)]
    else:
        w1, c1, w2b = tail_ws, tail_c.reshape(1, LAT), tail_wr
        out_specs, out_shape = [row, row, row], [out, out, out]
        mid_specs = [full, pl.BlockSpec((swidth, LAT), lambda i: (0, 0))]
        mid_args = [A, B_aug]
    res = pl.pallas_call(
        kern, grid=(grid,),
        in_specs=[row, s_spec] + mid_specs +
                 [vec, full, vec, vec, vec, full, vec, full],
        out_specs=out_specs, out_shape=out_shape,
    )(x, s_parts, *mid_args, c.reshape(1, LAT),
      wn2, bn2.reshape(1, LAT), scale.reshape(1, LAT), bias.reshape(1, LAT),
      w1, c1, w2b)
    return res[0] if final else res


def _sc_edge_step(with_h_out):
    mesh = plsc.VectorSubcoreMesh(core_axis_name="c", subcore_axis_name="s",
                                  num_cores=NC, num_subcores=NS)
    W = AUG if with_h_out else LAT
    s_out_t = jax.ShapeDtypeStruct((NC, N_PAD, W), _f32)
    if with_h_out:
        out_type = [jax.ShapeDtypeStruct((E, AUG), _f32), s_out_t]
    else:
        out_type = [s_out_t]
    slot_scratch = [
        pltpu.VMEM((C,), jnp.int32),
        pltpu.VMEM((C,), jnp.int32),
        pltpu.VMEM((C,), jnp.int32),
        pltpu.VMEM((C, LAT), _f32),
        pltpu.VMEM((C, LAT), _f32),
        pltpu.VMEM((C, W), _f32),
        pltpu.SemaphoreType.DMA,
        pltpu.SemaphoreType.DMA,
        pltpu.SemaphoreType.DMA,
        pltpu.SemaphoreType.DMA,
    ]
    if not with_h_out:
        slot_scratch += [pltpu.VMEM((C, LAT), _f32)]
    nslot = len(slot_scratch)
    scratch = slot_scratch * 2 + [
        pltpu.VMEM_SHARED((N_PAD, W), _f32),
    ]

    def body(*refs):
        if with_h_out:
            a_hbm, b_hbm, snd_hbm, rcv_hbm, h_out, s_out = refs[:6]
            hb_hbm = None
        else:
            a_hbm, b_hbm, snd_hbm, rcv_hbm, hb_hbm, s_out = refs[:6]
            h_out = None
        slots = [refs[6 + k * nslot: 6 + (k + 1) * nslot] for k in (0, 1)]
        acc = refs[6 + 2 * nslot]
        cid = lax.axis_index("c")
        sid = lax.axis_index("s")
        wid = cid * NS + sid
        row0 = sid * ROWS_PER_TILE

        zv = jnp.zeros((16,), _f32)
        ov = jnp.ones((16,), _f32)
        hbuf0 = slots[0][5]

        def fill_z(i, _):
            for j in range(W // 16):
                hbuf0[i, pl.ds(j * 16, 16)] = zv
            return 0
        lax.fori_loop(0, C, fill_z, 0)

        zcps = [pltpu.make_async_copy(hbuf0, acc.at[pl.ds(row0 + k * C, C)],
                                      slots[0][8])
                for k in range(ROWS_PER_TILE // C)]
        for cp in zcps:
            cp.start()
        for cp in zcps:
            cp.wait()

        if with_h_out:
            def fill_o(i, _):
                for hb in (slots[0][5], slots[1][5]):
                    hb[i, pl.ds(LAT, 16)] = ov
                return 0
            lax.fori_loop(0, C, fill_o, 0)
        plsc.subcore_barrier()

        def gather_cps(slot, k):
            cps = [pltpu.make_async_copy(a_hbm.at[slot[0]], slot[3], slot[6]),
                   pltpu.make_async_copy(b_hbm.at[slot[1]], slot[4], slot[7])]
            if not with_h_out:
                base = (wid + NW * k) * C
                cps.append(pltpu.make_async_copy(
                    hb_hbm.at[pl.ds(base, C)], slot[10], slot[9]))
            return cps

        def write_cps(slot, k):
            base = (wid + NW * k) * C
            cps = [pltpu.make_async_copy(slot[5], acc.at[slot[2]], slot[8])]
            if with_h_out:
                cps.append(pltpu.make_async_copy(
                    slot[5], h_out.at[pl.ds(base, C)], slot[9]))
            return cps

        def issue(slot, k):
            base = (wid + NW * k) * C
            pltpu.sync_copy(snd_hbm.at[pl.ds(base, C)], slot[0])
            pltpu.sync_copy(rcv_hbm.at[pl.ds(base, C)], slot[1])
            for cp in gather_cps(slot, k):
                cp.start()

        def compute(slot, k):
            sidx, ridx, ridx_sc, arows, brows, hbuf = slot[:6]
            for cp in gather_cps(slot, k):
                cp.wait()
            hbrows = None if with_h_out else slot[10]

            def row(rr, _):
                for j in range(LAT // 16):
                    sl = pl.ds(j * 16, 16)
                    acc_v = arows[rr, sl] + brows[rr, sl]
                    if not with_h_out:
                        acc_v = acc_v + hbrows[rr, sl]
                    hbuf[rr, sl] = jnp.maximum(acc_v, 0.0)
                return 0
            lax.fori_loop(0, C, row, 0)

            for off in (0, 16, 24):
                ridx_sc[pl.ds(off, 16)] = ridx[pl.ds(off, 16)]
            pltpu.async_copy(hbuf, acc.at[ridx_sc], slot[8], add=True)
            if with_h_out:
                base = (wid + NW * k) * C
                pltpu.async_copy(hbuf, h_out.at[pl.ds(base, C)], slot[9])

        def drain(slot, k):
            for cp in write_cps(slot, k):
                cp.wait()

        issue(slots[0], 0)
        issue(slots[1], 1)
        compute(slots[0], 0)
        issue(slots[0], 2)
        compute(slots[1], 1)
        issue(slots[1], 3)

        def pair(i, _):
            drain(slots[0], 2 * i - 2)
            compute(slots[0], 2 * i)
            issue(slots[0], 2 * i + 2)
            drain(slots[1], 2 * i - 1)
            compute(slots[1], 2 * i + 1)

            @pl.when(i < CPW // 2 - 1)
            def _():
                issue(slots[1], 2 * i + 3)
            return 0
        lax.fori_loop(1, CPW // 2, pair, 0)
        drain(slots[0], CPW - 3)
        compute(slots[0], CPW - 1)
        drain(slots[1], CPW - 2)
        drain(slots[0], CPW - 1)
        plsc.subcore_barrier()

        pltpu.sync_copy(acc.at[pl.ds(row0, ROWS_PER_TILE)],
                        s_out.at[cid, pl.ds(row0, ROWS_PER_TILE)])

    return pl.kernel(body, out_type=out_type, mesh=mesh, scratch_types=scratch,
                     compiler_params=pltpu.CompilerParams(use_tc_tiling_on_sc=False))


def kernel(nodes, globals_, senders, receivers, params):
    gv = globals_.reshape(1, -1).astype(_f32)

    (w1e0, b1e0), (w2e0, b2e0) = params["edge_mlp_0"]
    ws0, wr0, wg0 = w1e0[:LAT], w1e0[LAT:2 * LAT], w1e0[2 * LAT:]
    c0 = (gv @ wg0)[0] + b1e0

    (w1e1, b1e1), (w2e1, b2e1) = params["edge_mlp_1"]
    we1, ws1, wr1, wg1 = (w1e1[:LAT], w1e1[LAT:2 * LAT],
                          w1e1[2 * LAT:3 * LAT], w1e1[3 * LAT:])
    wec_pad = jnp.concatenate([w2e0 @ we1, jnp.zeros((AUG - LAT, LAT), _f32)], 0)
    c1 = b2e0 @ we1 + (gv @ wg1)[0] + b1e1

    def node_params(step, aug):
        (wn1, bn1), (wn2, bn2) = params["node_mlp_%d" % step]
        w2e, b2e = params["edge_mlp_%d" % step][1]
        wnr = wn1[LAT:2 * LAT]
        A = wn1[:LAT]
        B = w2e @ wnr
        v = b2e @ wnr
        if aug:
            B = jnp.concatenate(
                [B, v[None, :], jnp.zeros((AUG - LAT - 1, LAT), _f32)], 0)
        c = (gv @ wn1[2 * LAT:])[0] + bn1
        sc, bi = params["ln_%d" % step]
        return (A, B, c, wn2, bn2, sc, bi), v

    wemb, bemb = params["emb"]
    wdec, bdec = params["dec"]

    x0, a0, b0 = _tc_pre(nodes, wemb, bemb, ws0, c0, wr0)
    h0, s0 = _sc_edge_step(True)(a0, b0, senders, receivers)
    hb = _tc_edge_transform(h0, wec_pad)
    np0, _ = node_params(0, aug=True)
    x1, a1, b1 = _tc_node(x0, s0, *np0, tail_ws=ws1, tail_c=c1, tail_wr=wr1)
    (s1,) = _sc_edge_step(False)(a1, b1, senders, receivers, hb)
    np1, v1 = node_params(1, aug=False)
    out = _tc_node(x1, s1, *np1, dec_w=wdec, dec_b=bdec, d_parts=s0, v=v1)
    return out


# speedup vs baseline: 4.7347x; 1.6609x over previous
import functools

import jax
import jax.numpy as jnp
from jax import lax
from jax.experimental import pallas as pl
from jax.experimental.pallas import tpu as pltpu
from jax.experimental.pallas import tpu_sc as plsc

N = 10000
N_PAD = 10240
E = 160000
LAT = 128
NC = 2
NS = 16
NW = NC * NS
C = 40
NCHUNK = E // C
CPW = NCHUNK // NW
ROWS_PER_TILE = N_PAD // NS

_f32 = jnp.float32


def _mm(x, w):
    return jnp.dot(x, w, preferred_element_type=_f32)


def _tc_pre(nodes, wemb, bemb, ws, c_s, wr):
    blk, grid = 2000, N // 2000

    def kern(nref, wE, bE, wS, cS, wR, xo, ao, bo):
        xb = _mm(nref[...], wE[...]) + bE[...]
        xo[...] = xb
        ao[...] = _mm(xb, wS[...]) + cS[...]
        bo[...] = _mm(xb, wR[...])

    row = pl.BlockSpec((blk, LAT), lambda i: (i, 0))
    full = pl.BlockSpec((LAT, LAT), lambda i: (0, 0))
    vec = pl.BlockSpec((1, LAT), lambda i: (0, 0))
    out = jax.ShapeDtypeStruct((N, LAT), _f32)
    return pl.pallas_call(
        kern, grid=(grid,),
        in_specs=[pl.BlockSpec((blk, nodes.shape[1]), lambda i: (i, 0)),
                  pl.BlockSpec((nodes.shape[1], LAT), lambda i: (0, 0)),
                  vec, full, vec, full],
        out_specs=[row, row, row],
        out_shape=[out, out, out],
    )(nodes, wemb, bemb.reshape(1, LAT), ws, c_s.reshape(1, LAT), wr)


def _tc_edge_transform(h0, wec):
    blk, grid = 2000, E // 2000

    def kern(href, w, o):
        o[...] = _mm(href[...], w[...])

    return pl.pallas_call(
        kern, grid=(grid,),
        in_specs=[pl.BlockSpec((blk, LAT), lambda i: (i, 0)),
                  pl.BlockSpec((LAT, LAT), lambda i: (0, 0))],
        out_specs=pl.BlockSpec((blk, LAT), lambda i: (i, 0)),
        out_shape=jax.ShapeDtypeStruct((E, LAT), _f32),
    )(h0, wec)


def _tc_node(x, s_parts, d_parts, A, B, v, c, wn2, bn2, scale, bias,
             tail_ws=None, tail_c=None, tail_wr=None, dec_w=None, dec_b=None):
    blk, grid = 2000, N // 2000
    final = dec_w is not None

    def kern(xref, sref, dref, Ar, Br, vr, cr, w2r, b2r, scr, bir, w1r, c1r, w2b, *o):
        xb = xref[...]
        s = sref[0] + sref[1]
        dg = dref[0][:, 0:1] + dref[1][:, 0:1]
        hid = jnp.maximum(_mm(xb, Ar[...]) + _mm(s, Br[...]) + dg * vr[...]
                          + cr[...], 0.0)
        y = xb + _mm(hid, w2r[...]) + b2r[...]
        m = jnp.mean(y, axis=-1, keepdims=True)
        yc = y - m
        var = jnp.mean(yc * yc, axis=-1, keepdims=True)
        xn = yc * lax.rsqrt(var + 1e-6) * scr[...] + bir[...]
        if final:
            o[0][...] = _mm(xn, w1r[...]) + c1r[...]
        else:
            o[0][...] = xn
            o[1][...] = _mm(xn, w1r[...]) + c1r[...]
            o[2][...] = _mm(xn, w2b[...])

    row = pl.BlockSpec((blk, LAT), lambda i: (i, 0))
    full = pl.BlockSpec((LAT, LAT), lambda i: (0, 0))
    vec = pl.BlockSpec((1, LAT), lambda i: (0, 0))
    out = jax.ShapeDtypeStruct((N, LAT), _f32)
    if final:
        w1, c1, w2b = dec_w, dec_b.reshape(1, LAT), jnp.zeros((LAT, LAT), _f32)
        out_specs, out_shape = [row], [out]
    else:
        w1, c1, w2b = tail_ws, tail_c.reshape(1, LAT), tail_wr
        out_specs, out_shape = [row, row, row], [out, out, out]
    res = pl.pallas_call(
        kern, grid=(grid,),
        in_specs=[row,
                  pl.BlockSpec((2, blk, LAT), lambda i: (0, i, 0)),
                  pl.BlockSpec((2, blk, 16), lambda i: (0, i, 0)),
                  full, full, vec, vec, full, vec, vec, vec, full, vec, full],
        out_specs=out_specs, out_shape=out_shape,
    )(x, s_parts, d_parts, A, B, v.reshape(1, LAT), c.reshape(1, LAT),
      wn2, bn2.reshape(1, LAT), scale.reshape(1, LAT), bias.reshape(1, LAT),
      w1, c1, w2b)
    return res[0] if final else res


def _sc_edge_step(with_h_out):
    mesh = plsc.VectorSubcoreMesh(core_axis_name="c", subcore_axis_name="s",
                                  num_cores=NC, num_subcores=NS)
    s_out_t = jax.ShapeDtypeStruct((NC, N_PAD, LAT), _f32)
    if with_h_out:
        out_type = [jax.ShapeDtypeStruct((E, LAT), _f32), s_out_t,
                    jax.ShapeDtypeStruct((NC, N_PAD, 16), _f32)]
    else:
        out_type = [s_out_t]
    slot_scratch = [
        pltpu.VMEM((C,), jnp.int32),
        pltpu.VMEM((C,), jnp.int32),
        pltpu.VMEM((C,), jnp.int32),
        pltpu.VMEM((C, LAT), _f32),
        pltpu.VMEM((C, LAT), _f32),
        pltpu.VMEM((C, LAT), _f32),
        pltpu.SemaphoreType.DMA,
        pltpu.SemaphoreType.DMA,
        pltpu.SemaphoreType.DMA,
        pltpu.SemaphoreType.DMA,
        pltpu.SemaphoreType.DMA,
    ]
    if not with_h_out:
        slot_scratch += [pltpu.VMEM((C, LAT), _f32)]
    nslot = len(slot_scratch)
    scratch = slot_scratch * 2 + [
        pltpu.VMEM((C, 16), _f32),
        pltpu.VMEM_SHARED((N_PAD, LAT), _f32),
        pltpu.VMEM_SHARED((N_PAD, 16), _f32),
    ]

    def body(*refs):
        if with_h_out:
            a_hbm, b_hbm, snd_hbm, rcv_hbm, h_out, s_out, d_out = refs[:7]
            hb_hbm = None
            off = 7
        else:
            a_hbm, b_hbm, snd_hbm, rcv_hbm, hb_hbm, s_out = refs[:6]
            h_out = d_out = None
            off = 6
        slots = [refs[off + k * nslot: off + (k + 1) * nslot] for k in (0, 1)]
        ones16 = refs[off + 2 * nslot]
        acc = refs[off + 2 * nslot + 1]
        dacc = refs[off + 2 * nslot + 2]
        cid = lax.axis_index("c")
        sid = lax.axis_index("s")
        wid = cid * NS + sid
        row0 = sid * ROWS_PER_TILE

        zv = jnp.zeros((16,), _f32)
        ov = jnp.ones((16,), _f32)
        hbuf0 = slots[0][5]

        def fill_z(i, _):
            for j in range(LAT // 16):
                hbuf0[i, pl.ds(j * 16, 16)] = zv
            ones16[i, pl.ds(0, 16)] = zv
            return 0
        lax.fori_loop(0, C, fill_z, 0)

        zcps = [pltpu.make_async_copy(hbuf0, acc.at[pl.ds(row0 + k * C, C)],
                                      slots[0][8])
                for k in range(ROWS_PER_TILE // C)]
        if with_h_out:
            zcps += [pltpu.make_async_copy(
                ones16, dacc.at[pl.ds(row0 + k * C, C)], slots[0][10])
                for k in range(ROWS_PER_TILE // C)]
        for cp in zcps:
            cp.start()
        for cp in zcps:
            cp.wait()

        if with_h_out:
            def fill_o(i, _):
                ones16[i, pl.ds(0, 16)] = ov
                return 0
            lax.fori_loop(0, C, fill_o, 0)
        plsc.subcore_barrier()

        def gather_cps(slot, k):
            cps = [pltpu.make_async_copy(a_hbm.at[slot[0]], slot[3], slot[6]),
                   pltpu.make_async_copy(b_hbm.at[slot[1]], slot[4], slot[7])]
            if not with_h_out:
                base = (wid + NW * k) * C
                cps.append(pltpu.make_async_copy(
                    hb_hbm.at[pl.ds(base, C)], slot[11], slot[9]))
            return cps

        def write_cps(slot, k):
            base = (wid + NW * k) * C
            cps = [pltpu.make_async_copy(slot[5], acc.at[slot[2]], slot[8])]
            if with_h_out:
                cps.append(pltpu.make_async_copy(
                    slot[5], h_out.at[pl.ds(base, C)], slot[9]))
                cps.append(pltpu.make_async_copy(
                    ones16, dacc.at[slot[2]], slot[10]))
            return cps

        def issue(slot, k):
            base = (wid + NW * k) * C
            pltpu.sync_copy(snd_hbm.at[pl.ds(base, C)], slot[0])
            pltpu.sync_copy(rcv_hbm.at[pl.ds(base, C)], slot[1])
            for cp in gather_cps(slot, k):
                cp.start()

        def compute(slot, k):
            sidx, ridx, ridx_sc, arows, brows, hbuf = slot[:6]
            for cp in gather_cps(slot, k):
                cp.wait()
            hbrows = None if with_h_out else slot[11]

            def row(rr, _):
                for j in range(LAT // 16):
                    sl = pl.ds(j * 16, 16)
                    acc_v = arows[rr, sl] + brows[rr, sl]
                    if not with_h_out:
                        acc_v = acc_v + hbrows[rr, sl]
                    hbuf[rr, sl] = jnp.maximum(acc_v, 0.0)
                return 0
            lax.fori_loop(0, C, row, 0)

            for goff in (0, 16, 24):
                ridx_sc[pl.ds(goff, 16)] = ridx[pl.ds(goff, 16)]
            pltpu.async_copy(hbuf, acc.at[ridx_sc], slot[8], add=True)
            if with_h_out:
                base = (wid + NW * k) * C
                pltpu.async_copy(hbuf, h_out.at[pl.ds(base, C)], slot[9])
                pltpu.async_copy(ones16, dacc.at[ridx_sc], slot[10], add=True)

        def drain(slot, k):
            for cp in write_cps(slot, k):
                cp.wait()

        issue(slots[0], 0)
        issue(slots[1], 1)
        compute(slots[0], 0)
        issue(slots[0], 2)
        compute(slots[1], 1)
        issue(slots[1], 3)

        def pair(i, _):
            drain(slots[0], 2 * i - 2)
            compute(slots[0], 2 * i)
            issue(slots[0], 2 * i + 2)
            drain(slots[1], 2 * i - 1)
            compute(slots[1], 2 * i + 1)

            @pl.when(i < CPW // 2 - 1)
            def _():
                issue(slots[1], 2 * i + 3)
            return 0
        lax.fori_loop(1, CPW // 2, pair, 0)
        drain(slots[0], CPW - 3)
        compute(slots[0], CPW - 1)
        drain(slots[1], CPW - 2)
        drain(slots[0], CPW - 1)
        plsc.subcore_barrier()

        pltpu.sync_copy(acc.at[pl.ds(row0, ROWS_PER_TILE)],
                        s_out.at[cid, pl.ds(row0, ROWS_PER_TILE)])
        if with_h_out:
            pltpu.sync_copy(dacc.at[pl.ds(row0, ROWS_PER_TILE)],
                            d_out.at[cid, pl.ds(row0, ROWS_PER_TILE)])

    return pl.kernel(body, out_type=out_type, mesh=mesh, scratch_types=scratch,
                     compiler_params=pltpu.CompilerParams(use_tc_tiling_on_sc=False))


def kernel(nodes, globals_, senders, receivers, params):
    gv = globals_.reshape(1, -1).astype(_f32)

    (w1e0, b1e0), (w2e0, b2e0) = params["edge_mlp_0"]
    ws0, wr0, wg0 = w1e0[:LAT], w1e0[LAT:2 * LAT], w1e0[2 * LAT:]
    c0 = (gv @ wg0)[0] + b1e0

    (w1e1, b1e1), (w2e1, b2e1) = params["edge_mlp_1"]
    we1, ws1, wr1, wg1 = (w1e1[:LAT], w1e1[LAT:2 * LAT],
                          w1e1[2 * LAT:3 * LAT], w1e1[3 * LAT:])
    wec = w2e0 @ we1
    c1 = b2e0 @ we1 + (gv @ wg1)[0] + b1e1

    def node_params(step):
        (wn1, bn1), (wn2, bn2) = params["node_mlp_%d" % step]
        w2e, b2e = params["edge_mlp_%d" % step][1]
        wnr = wn1[LAT:2 * LAT]
        A = wn1[:LAT]
        B = w2e @ wnr
        v = b2e @ wnr
        c = (gv @ wn1[2 * LAT:])[0] + bn1
        sc, bi = params["ln_%d" % step]
        return A, B, v, c, wn2, bn2, sc, bi

    wemb, bemb = params["emb"]
    wdec, bdec = params["dec"]

    x0, a0, b0 = _tc_pre(nodes, wemb, bemb, ws0, c0, wr0)
    h0, s0, d0 = _sc_edge_step(True)(a0, b0, senders, receivers)
    hb = _tc_edge_transform(h0, wec)
    x1, a1, b1 = _tc_node(x0, s0, d0, *node_params(0),
                          tail_ws=ws1, tail_c=c1, tail_wr=wr1)
    (s1,) = _sc_edge_step(False)(a1, b1, senders, receivers, hb)
    out = _tc_node(x1, s1, d0, *node_params(1), dec_w=wdec, dec_b=bdec)
    return out
